# baseline (speedup 1.0000x reference)
import functools

import jax
import jax.numpy as jnp
from jax import lax
from jax.experimental import pallas as pl
from jax.experimental.pallas import tpu as pltpu
from jax.experimental.pallas import tpu_sc as plsc

_VOCAB = 100000
_D = 64
_BAG = 200
_CHUNK = 8
_NCHUNKS = _BAG // _CHUNK
_BLK = 2000
_NBLK = _VOCAB // _BLK
_SHIFT = 32.0


def _sc_bag_partials(word_bag, embedding_weight):
    info = plsc.get_sparse_core_info()
    num_workers = info.num_cores * info.num_subcores

    @functools.partial(
        pl.kernel,
        mesh=plsc.VectorSubcoreMesh(core_axis_name="c", subcore_axis_name="s"),
        out_type=jax.ShapeDtypeStruct((num_workers, _D), jnp.float32),
        scratch_types=[
            pltpu.VMEM((_CHUNK,), jnp.int32),
            pltpu.VMEM((_CHUNK, _D), jnp.float32),
            pltpu.VMEM((_D,), jnp.float32),
            pltpu.SemaphoreType.DMA,
        ],
        compiler_params=pltpu.CompilerParams(use_tc_tiling_on_sc=False),
    )
    def gather_kernel(idx_hbm, table_hbm, out_hbm, idx_v, rows_v, part_v, sem):
        wid = lax.axis_index("s") * info.num_cores + lax.axis_index("c")

        @pl.when(wid < _NCHUNKS)
        def _active():
            pltpu.sync_copy(idx_hbm.at[pl.ds(wid * _CHUNK, _CHUNK)], idx_v)
            pltpu.async_copy(table_hbm.at[idx_v], rows_v, sem).wait()
            for c in range(_D // 16):
                acc = rows_v[0, pl.ds(c * 16, 16)]
                for r in range(1, _CHUNK):
                    acc = acc + rows_v[r, pl.ds(c * 16, 16)]
                part_v[pl.ds(c * 16, 16)] = acc

        @pl.when(wid >= _NCHUNKS)
        def _idle():
            for c in range(_D // 16):
                part_v[pl.ds(c * 16, 16)] = jnp.zeros((16,), jnp.float32)

        pltpu.sync_copy(part_v, out_hbm.at[wid])

    return gather_kernel(word_bag, embedding_weight)


def _tc_project_softmax(partials, weight, bias_3d):
    _SLOT = 2048

    def body(p_ref, w_ref, b_ref, o_ref, s_ref, e_ref):
        i = pl.program_id(0)
        bag_sum = jnp.sum(p_ref[...], axis=0, keepdims=True)
        logits = lax.dot_general(
            bag_sum, w_ref[...], (((1,), (1,)), ((), ())),
            preferred_element_type=jnp.float32)
        e = jnp.exp(logits * (1.0 / _BAG) + b_ref[0] - _SHIFT)
        e_ref[:, pl.ds(i * _SLOT, _BLK)] = e

        @pl.when(i == 0)
        def _init():
            s_ref[0] = 0.0

        s_ref[0] += jnp.sum(e)

        @pl.when(i == _NBLK - 1)
        def _normalize():
            inv = 1.0 / s_ref[0]
            for j in range(_NBLK):
                o_ref[:, j * _BLK:(j + 1) * _BLK] = (
                    e_ref[:, j * _SLOT:j * _SLOT + _BLK] * inv)

    return pl.pallas_call(
        body,
        grid=(_NBLK,),
        in_specs=[
            pl.BlockSpec(partials.shape, lambda i: (0, 0)),
            pl.BlockSpec((_BLK, _D), lambda i: (i, 0)),
            pl.BlockSpec((1, 1, _BLK), lambda i: (i, 0, 0)),
        ],
        out_specs=pl.BlockSpec((1, _VOCAB), lambda i: (0, 0)),
        out_shape=jax.ShapeDtypeStruct((1, _VOCAB), jnp.float32),
        scratch_shapes=[pltpu.SMEM((1,), jnp.float32),
                        pltpu.VMEM((1, _NBLK * _SLOT), jnp.float32)],
        compiler_params=pltpu.CompilerParams(
            dimension_semantics=("arbitrary",)),
    )(partials, weight, bias_3d)


def kernel(wordBag, embedding_weight, rebound_weight, rebound_bias):
    partials = _sc_bag_partials(wordBag, embedding_weight)
    bias_3d = rebound_bias.reshape(_NBLK, 1, _BLK)
    return _tc_project_softmax(partials, rebound_weight, bias_3d)


# speedup vs baseline: 1.3354x; 1.3354x over previous
import jax
import jax.numpy as jnp
from jax import lax
from jax.experimental import pallas as pl
from jax.experimental.pallas import tpu as pltpu

_VOCAB = 100000
_D = 64
_BAG = 200
_BLK = 2000
_NBLK = _VOCAB // _BLK
_SLOT = 2048
_SHIFT = 32.0


def _body(idx_ref, tbl_ref, w_ref, b_ref, o_ref,
          rows_v, bag_v, s_ref, e_ref, sem):
    i = pl.program_id(0)

    @pl.when(i == 0)
    def _gather_and_pool():
        copies = [
            pltpu.make_async_copy(
                tbl_ref.at[pl.ds(idx_ref[j], 1)],
                rows_v.at[pl.ds(j, 1)], sem)
            for j in range(_BAG)
        ]
        for c in copies:
            c.start()
        for c in copies:
            c.wait()
        bag_v[...] = jnp.sum(rows_v[...], axis=0, keepdims=True)
        s_ref[0] = 0.0

    logits = lax.dot_general(
        bag_v[...], w_ref[...], (((1,), (1,)), ((), ())),
        preferred_element_type=jnp.float32)
    e = jnp.exp(logits * (1.0 / _BAG) + b_ref[0] - _SHIFT)
    e_ref[:, pl.ds(i * _SLOT, _BLK)] = e
    s_ref[0] += jnp.sum(e)

    @pl.when(i == _NBLK - 1)
    def _normalize():
        inv = 1.0 / s_ref[0]
        for j in range(_NBLK):
            o_ref[:, j * _BLK:(j + 1) * _BLK] = (
                e_ref[:, j * _SLOT:j * _SLOT + _BLK] * inv)


def kernel(wordBag, embedding_weight, rebound_weight, rebound_bias):
    bias_3d = rebound_bias.reshape(_NBLK, 1, _BLK)
    grid_spec = pltpu.PrefetchScalarGridSpec(
        num_scalar_prefetch=1,
        grid=(_NBLK,),
        in_specs=[
            pl.BlockSpec(memory_space=pl.ANY),
            pl.BlockSpec((_BLK, _D), lambda i, idx: (i, 0)),
            pl.BlockSpec((1, 1, _BLK), lambda i, idx: (i, 0, 0)),
        ],
        out_specs=pl.BlockSpec((1, _VOCAB), lambda i, idx: (0, 0)),
        scratch_shapes=[
            pltpu.VMEM((_BAG, _D), jnp.float32),
            pltpu.VMEM((1, _D), jnp.float32),
            pltpu.SMEM((1,), jnp.float32),
            pltpu.VMEM((1, _NBLK * _SLOT), jnp.float32),
            pltpu.SemaphoreType.DMA,
        ],
    )
    return pl.pallas_call(
        _body,
        grid_spec=grid_spec,
        out_shape=jax.ShapeDtypeStruct((1, _VOCAB), jnp.float32),
        compiler_params=pltpu.CompilerParams(
            dimension_semantics=("arbitrary",)),
    )(wordBag, embedding_weight, rebound_weight, bias_3d)
